# Initial kernel scaffold; baseline (speedup 1.0000x reference)
import jax
import jax.numpy as jnp
from jax.experimental import pallas as pl


def kernel(x, edge_index, W_l1, W_r1, b1, W_l2, W_r2, b2):
    raise NotImplementedError("write your pallas kernel here")


# speedup vs baseline: 9.7776x; 9.7776x over previous
import functools

import jax
import jax.numpy as jnp
from jax import lax
from jax.experimental import pallas as pl
from jax.experimental.pallas import tpu as pltpu
from jax.experimental.pallas import tpu_sc as plsc

_N = 10000
_E = 320000
_D = 128
_H = 64
_Z = 32

_NC = 2
_NS = 16
_NW = _NC * _NS
_EPW = _E // _NW
_C = 80
_NCH = _EPW // _C
_RPT = 624
_TAIL = _N - _RPT * _NS

_DW = 8


def _sc_agg(y, src3, dst3, width, with_deg):
    mesh = plsc.VectorSubcoreMesh(core_axis_name="c", subcore_axis_name="s")

    out_type = [jax.ShapeDtypeStruct((_NC * _N, width), jnp.float32)]
    if with_deg:
        out_type.append(jax.ShapeDtypeStruct((_NC * _N, _DW), jnp.float32))

    scratch = [
        pltpu.VMEM((_NCH, _C), jnp.int32),
        pltpu.VMEM((_NCH, _C), jnp.int32),
        pltpu.VMEM((_C, width), jnp.float32),
        pltpu.VMEM_SHARED((_N, width), jnp.float32),
        pltpu.SemaphoreType.DMA,
    ]
    if with_deg:
        scratch += [
            pltpu.VMEM((_C, _DW), jnp.float32),
            pltpu.VMEM_SHARED((_N, _DW), jnp.float32),
        ]

    def body(y_hbm, src_hbm, dst_hbm, zsum_hbm, zdeg_hbm, ones_hbm,
             *refs):
        if with_deg:
            (out_hbm, deg_hbm, src_v, dst_v, buf, acc, sem,
             ones_v, dacc) = refs
        else:
            (out_hbm, src_v, dst_v, buf, acc, sem) = refs
        cid = lax.axis_index("c")
        sid = lax.axis_index("s")
        wid = cid * _NS + sid

        r0 = sid * _RPT
        pltpu.sync_copy(zsum_hbm.at[pl.ds(r0, _RPT)], acc.at[pl.ds(r0, _RPT)])
        if with_deg:
            pltpu.sync_copy(zdeg_hbm.at[pl.ds(r0, _RPT)],
                            dacc.at[pl.ds(r0, _RPT)])
            pltpu.sync_copy(ones_hbm, ones_v)

        @pl.when(sid == 0)
        def _zero_tail():
            t0 = _RPT * _NS
            pltpu.sync_copy(zsum_hbm.at[pl.ds(t0, _TAIL)],
                            acc.at[pl.ds(t0, _TAIL)])
            if with_deg:
                pltpu.sync_copy(zdeg_hbm.at[pl.ds(t0, _TAIL)],
                                dacc.at[pl.ds(t0, _TAIL)])
        pltpu.sync_copy(src_hbm.at[wid], src_v)
        pltpu.sync_copy(dst_hbm.at[wid], dst_v)
        plsc.subcore_barrier()

        @pl.loop(0, _NCH)
        def chunk(i):
            pltpu.async_copy(y_hbm.at[src_v.at[i]], buf, sem).wait()
            pltpu.sync_copy(buf, acc.at[dst_v.at[i]], add=True)
            if with_deg:
                pltpu.sync_copy(ones_v, dacc.at[dst_v.at[i]], add=True)

        plsc.subcore_barrier()
        o0 = cid * _N + sid * _RPT
        pltpu.sync_copy(acc.at[pl.ds(r0, _RPT)], out_hbm.at[pl.ds(o0, _RPT)])
        if with_deg:
            pltpu.sync_copy(dacc.at[pl.ds(r0, _RPT)],
                            deg_hbm.at[pl.ds(o0, _RPT)])

        @pl.when(sid == 0)
        def _copy_tail():
            t0 = _RPT * _NS
            pltpu.sync_copy(acc.at[pl.ds(t0, _TAIL)],
                            out_hbm.at[pl.ds(cid * _N + t0, _TAIL)])
            if with_deg:
                pltpu.sync_copy(dacc.at[pl.ds(t0, _TAIL)],
                                deg_hbm.at[pl.ds(cid * _N + t0, _TAIL)])

    zsum = jnp.zeros((_N, width), jnp.float32)
    zdeg = jnp.zeros((_N, _DW), jnp.float32)
    ones = jnp.ones((_C, _DW), jnp.float32)
    k = pl.kernel(body, out_type=out_type, mesh=mesh, scratch_types=scratch,
                  compiler_params=pltpu.CompilerParams(
                      use_tc_tiling_on_sc=False))
    res = k(y, src3, dst3, zsum, zdeg, ones)
    if with_deg:
        psum, pdeg = res
        return (psum.reshape(_NC, _N, width), pdeg.reshape(_NC, _N, _DW))
    return res[0].reshape(_NC, _N, width)


_BN = 2000


def _tc1_body(x_ref, wl_ref, wr_ref, y1_ref, r1_ref):
    xb = x_ref[...]
    y1_ref[...] = jnp.dot(xb, wl_ref[...], preferred_element_type=jnp.float32)
    r1_ref[...] = jnp.dot(xb, wr_ref[...], preferred_element_type=jnp.float32)


def _tc2_body(sum_ref, deg_ref, r1_ref, b1_ref, wl2_ref, wr2_ref,
              y2_ref, r2_ref):
    s = sum_ref[0] + sum_ref[1]
    deg = deg_ref[0, :, 0:1] + deg_ref[1, :, 0:1]
    h = jnp.maximum(s / jnp.maximum(deg, 1.0) + r1_ref[...] + b1_ref[...], 0.0)
    y2_ref[...] = jnp.dot(h, wl2_ref[...], preferred_element_type=jnp.float32)
    r2_ref[...] = jnp.dot(h, wr2_ref[...], preferred_element_type=jnp.float32)


def _tc3_body(sum_ref, deg_ref, r2_ref, b2_ref, z_ref):
    s = sum_ref[0] + sum_ref[1]
    deg = deg_ref[0, :, 0:1] + deg_ref[1, :, 0:1]
    z_ref[...] = jnp.maximum(
        s / jnp.maximum(deg, 1.0) + r2_ref[...] + b2_ref[...], 0.0)


def kernel(x, edge_index, W_l1, W_r1, b1, W_l2, W_r2, b2):
    grid = _N // _BN
    src3 = edge_index[0].reshape(_NW, _NCH, _C)
    dst3 = edge_index[1].reshape(_NW, _NCH, _C)

    y1, r1 = pl.pallas_call(
        _tc1_body,
        grid=(grid,),
        in_specs=[
            pl.BlockSpec((_BN, _D), lambda i: (i, 0)),
            pl.BlockSpec((_D, _H), lambda i: (0, 0)),
            pl.BlockSpec((_D, _H), lambda i: (0, 0)),
        ],
        out_specs=[
            pl.BlockSpec((_BN, _H), lambda i: (i, 0)),
            pl.BlockSpec((_BN, _H), lambda i: (i, 0)),
        ],
        out_shape=[
            jax.ShapeDtypeStruct((_N, _H), jnp.float32),
            jax.ShapeDtypeStruct((_N, _H), jnp.float32),
        ],
    )(x, W_l1, W_r1)

    psum1, pdeg = _sc_agg(y1, src3, dst3, _H, with_deg=True)

    y2, r2 = pl.pallas_call(
        _tc2_body,
        grid=(grid,),
        in_specs=[
            pl.BlockSpec((_NC, _BN, _H), lambda i: (0, i, 0)),
            pl.BlockSpec((_NC, _BN, _DW), lambda i: (0, i, 0)),
            pl.BlockSpec((_BN, _H), lambda i: (i, 0)),
            pl.BlockSpec((1, _H), lambda i: (0, 0)),
            pl.BlockSpec((_H, _Z), lambda i: (0, 0)),
            pl.BlockSpec((_H, _Z), lambda i: (0, 0)),
        ],
        out_specs=[
            pl.BlockSpec((_BN, _Z), lambda i: (i, 0)),
            pl.BlockSpec((_BN, _Z), lambda i: (i, 0)),
        ],
        out_shape=[
            jax.ShapeDtypeStruct((_N, _Z), jnp.float32),
            jax.ShapeDtypeStruct((_N, _Z), jnp.float32),
        ],
    )(psum1, pdeg, r1, b1.reshape(1, _H), W_l2, W_r2)

    psum2 = _sc_agg(y2, src3, dst3, _Z, with_deg=False)

    z = pl.pallas_call(
        _tc3_body,
        grid=(grid,),
        in_specs=[
            pl.BlockSpec((_NC, _BN, _Z), lambda i: (0, i, 0)),
            pl.BlockSpec((_NC, _BN, _DW), lambda i: (0, i, 0)),
            pl.BlockSpec((_BN, _Z), lambda i: (i, 0)),
            pl.BlockSpec((1, _Z), lambda i: (0, 0)),
        ],
        out_specs=pl.BlockSpec((_BN, _Z), lambda i: (i, 0)),
        out_shape=jax.ShapeDtypeStruct((_N, _Z), jnp.float32),
    )(psum2, pdeg, r2, b2.reshape(1, _Z))

    return z


# speedup vs baseline: 17.8771x; 1.8284x over previous
import functools

import jax
import jax.numpy as jnp
from jax import lax
from jax.experimental import pallas as pl
from jax.experimental.pallas import tpu as pltpu
from jax.experimental.pallas import tpu_sc as plsc

_N = 10000
_E = 320000
_D = 128
_H = 64
_Z = 32

_NC = 2
_NS = 16
_NW = _NC * _NS
_EPW = _E // _NW
_C = 125
_NCH = _EPW // _C
_NSLOT = 4
_RPT = 624
_TAIL = _N - _RPT * _NS

_DW = 8


def _sc_agg(y, src3, dst3, width, with_deg):
    mesh = plsc.VectorSubcoreMesh(core_axis_name="c", subcore_axis_name="s")

    out_type = [jax.ShapeDtypeStruct((_NC * _N, width), jnp.float32)]
    if with_deg:
        out_type.append(jax.ShapeDtypeStruct((_NC * _N, _DW), jnp.float32))

    scratch = [
        pltpu.VMEM((_NCH, _C), jnp.int32),
        pltpu.VMEM((_NCH, _C), jnp.int32),
        [pltpu.VMEM((_C, width), jnp.float32)] * _NSLOT,
        pltpu.VMEM_SHARED((_N, width), jnp.float32),
        [pltpu.SemaphoreType.DMA] * _NSLOT,
        [pltpu.SemaphoreType.DMA] * _NSLOT,
    ]
    if with_deg:
        scratch += [
            pltpu.VMEM((_C, _DW), jnp.float32),
            pltpu.VMEM_SHARED((_N, _DW), jnp.float32),
        ]

    def body(y_hbm, src_hbm, dst_hbm, zsum_hbm, zdeg_hbm, ones_hbm,
             *refs):
        if with_deg:
            (out_hbm, deg_hbm, src_v, dst_v, bufs, acc, gsem, ssem,
             ones_v, dacc) = refs
        else:
            (out_hbm, src_v, dst_v, bufs, acc, gsem, ssem) = refs
            ones_v = dacc = None
        cid = lax.axis_index("c")
        sid = lax.axis_index("s")
        wid = cid * _NS + sid

        r0 = sid * _RPT
        pltpu.sync_copy(zsum_hbm.at[pl.ds(r0, _RPT)], acc.at[pl.ds(r0, _RPT)])
        if with_deg:
            pltpu.sync_copy(zdeg_hbm.at[pl.ds(r0, _RPT)],
                            dacc.at[pl.ds(r0, _RPT)])
            pltpu.sync_copy(ones_hbm, ones_v)

        @pl.when(sid == 0)
        def _zero_tail():
            t0 = _RPT * _NS
            pltpu.sync_copy(zsum_hbm.at[pl.ds(t0, _TAIL)],
                            acc.at[pl.ds(t0, _TAIL)])
            if with_deg:
                pltpu.sync_copy(zdeg_hbm.at[pl.ds(t0, _TAIL)],
                                dacc.at[pl.ds(t0, _TAIL)])
        pltpu.sync_copy(src_hbm.at[wid], src_v)
        pltpu.sync_copy(dst_hbm.at[wid], dst_v)
        plsc.subcore_barrier()

        def issue_gather(k, j):
            pltpu.async_copy(y_hbm.at[src_v.at[k]], bufs[j], gsem[j])

        def issue_scatter(k, j):
            pltpu.async_copy(bufs[j], acc.at[dst_v.at[k]], ssem[j],
                             add=True)
            if with_deg:
                pltpu.async_copy(ones_v, dacc.at[dst_v.at[k]], ssem[j],
                                 add=True)

        def wait_gather(j):
            pltpu.make_async_copy(y_hbm.at[pl.ds(0, _C)], bufs[j],
                                  gsem[j]).wait()

        def wait_scatter(j):
            pltpu.make_async_copy(y_hbm.at[pl.ds(0, _C)], bufs[j],
                                  ssem[j]).wait()
            if with_deg:
                pltpu.make_async_copy(ones_hbm, ones_v, ssem[j]).wait()

        for j in range(_NSLOT):
            issue_gather(j, j)

        @pl.loop(0, (_NCH - _NSLOT) // _NSLOT)
        def group(h):
            base = h * _NSLOT
            for bank in (0, 1):
                for t in (0, 1):
                    j = 2 * bank + t
                    wait_gather(j)
                    issue_scatter(base + j, j)
                for t in (0, 1):
                    j = 2 * bank + t
                    wait_scatter(j)
                    issue_gather(base + _NSLOT + j, j)

        for j in range(_NSLOT):
            wait_gather(j)
            issue_scatter(_NCH - _NSLOT + j, j)
        for j in range(_NSLOT):
            wait_scatter(j)

        plsc.subcore_barrier()
        o0 = cid * _N + sid * _RPT
        pltpu.sync_copy(acc.at[pl.ds(r0, _RPT)], out_hbm.at[pl.ds(o0, _RPT)])
        if with_deg:
            pltpu.sync_copy(dacc.at[pl.ds(r0, _RPT)],
                            deg_hbm.at[pl.ds(o0, _RPT)])

        @pl.when(sid == 0)
        def _copy_tail():
            t0 = _RPT * _NS
            pltpu.sync_copy(acc.at[pl.ds(t0, _TAIL)],
                            out_hbm.at[pl.ds(cid * _N + t0, _TAIL)])
            if with_deg:
                pltpu.sync_copy(dacc.at[pl.ds(t0, _TAIL)],
                                deg_hbm.at[pl.ds(cid * _N + t0, _TAIL)])

    zsum = jnp.zeros((_N, width), jnp.float32)
    zdeg = jnp.zeros((_N, _DW), jnp.float32)
    ones = jnp.ones((_C, _DW), jnp.float32)
    k = pl.kernel(body, out_type=out_type, mesh=mesh, scratch_types=scratch,
                  compiler_params=pltpu.CompilerParams(
                      use_tc_tiling_on_sc=False))
    res = k(y, src3, dst3, zsum, zdeg, ones)
    if with_deg:
        psum, pdeg = res
        return (psum.reshape(_NC, _N, width), pdeg.reshape(_NC, _N, _DW))
    return res[0].reshape(_NC, _N, width)


_BN = 2000


def _tc1_body(x_ref, wl_ref, wr_ref, y1_ref, r1_ref):
    xb = x_ref[...]
    y1_ref[...] = jnp.dot(xb, wl_ref[...], preferred_element_type=jnp.float32)
    r1_ref[...] = jnp.dot(xb, wr_ref[...], preferred_element_type=jnp.float32)


def _tc2_body(sum_ref, deg_ref, r1_ref, b1_ref, wl2_ref, wr2_ref,
              y2_ref, r2_ref):
    s = sum_ref[0] + sum_ref[1]
    deg = deg_ref[0, :, 0:1] + deg_ref[1, :, 0:1]
    h = jnp.maximum(s / jnp.maximum(deg, 1.0) + r1_ref[...] + b1_ref[...], 0.0)
    y2_ref[...] = jnp.dot(h, wl2_ref[...], preferred_element_type=jnp.float32)
    r2_ref[...] = jnp.dot(h, wr2_ref[...], preferred_element_type=jnp.float32)


def _tc3_body(sum_ref, deg_ref, r2_ref, b2_ref, z_ref):
    s = sum_ref[0] + sum_ref[1]
    deg = deg_ref[0, :, 0:1] + deg_ref[1, :, 0:1]
    z_ref[...] = jnp.maximum(
        s / jnp.maximum(deg, 1.0) + r2_ref[...] + b2_ref[...], 0.0)


def kernel(x, edge_index, W_l1, W_r1, b1, W_l2, W_r2, b2):
    grid = _N // _BN
    src3 = edge_index[0].reshape(_NW, _NCH, _C)
    dst3 = edge_index[1].reshape(_NW, _NCH, _C)

    y1, r1 = pl.pallas_call(
        _tc1_body,
        grid=(grid,),
        in_specs=[
            pl.BlockSpec((_BN, _D), lambda i: (i, 0)),
            pl.BlockSpec((_D, _H), lambda i: (0, 0)),
            pl.BlockSpec((_D, _H), lambda i: (0, 0)),
        ],
        out_specs=[
            pl.BlockSpec((_BN, _H), lambda i: (i, 0)),
            pl.BlockSpec((_BN, _H), lambda i: (i, 0)),
        ],
        out_shape=[
            jax.ShapeDtypeStruct((_N, _H), jnp.float32),
            jax.ShapeDtypeStruct((_N, _H), jnp.float32),
        ],
    )(x, W_l1, W_r1)

    psum1, pdeg = _sc_agg(y1, src3, dst3, _H, with_deg=True)

    y2, r2 = pl.pallas_call(
        _tc2_body,
        grid=(grid,),
        in_specs=[
            pl.BlockSpec((_NC, _BN, _H), lambda i: (0, i, 0)),
            pl.BlockSpec((_NC, _BN, _DW), lambda i: (0, i, 0)),
            pl.BlockSpec((_BN, _H), lambda i: (i, 0)),
            pl.BlockSpec((1, _H), lambda i: (0, 0)),
            pl.BlockSpec((_H, _Z), lambda i: (0, 0)),
            pl.BlockSpec((_H, _Z), lambda i: (0, 0)),
        ],
        out_specs=[
            pl.BlockSpec((_BN, _Z), lambda i: (i, 0)),
            pl.BlockSpec((_BN, _Z), lambda i: (i, 0)),
        ],
        out_shape=[
            jax.ShapeDtypeStruct((_N, _Z), jnp.float32),
            jax.ShapeDtypeStruct((_N, _Z), jnp.float32),
        ],
    )(psum1, pdeg, r1, b1.reshape(1, _H), W_l2, W_r2)

    psum2 = _sc_agg(y2, src3, dst3, _Z, with_deg=False)

    z = pl.pallas_call(
        _tc3_body,
        grid=(grid,),
        in_specs=[
            pl.BlockSpec((_NC, _BN, _Z), lambda i: (0, i, 0)),
            pl.BlockSpec((_NC, _BN, _DW), lambda i: (0, i, 0)),
            pl.BlockSpec((_BN, _Z), lambda i: (i, 0)),
            pl.BlockSpec((1, _Z), lambda i: (0, 0)),
        ],
        out_specs=pl.BlockSpec((_BN, _Z), lambda i: (i, 0)),
        out_shape=jax.ShapeDtypeStruct((_N, _Z), jnp.float32),
    )(psum2, pdeg, r2, b2.reshape(1, _Z))

    return z
